# Initial kernel scaffold; baseline (speedup 1.0000x reference)
import functools

import jax
import jax.numpy as jnp
from jax import lax
from jax.experimental import pallas as pl
from jax.experimental.pallas import tpu as pltpu
from jax.experimental.pallas import tpu_sc as plsc

_NSUB = 16
_NCORES = 2
_LANES = 16


def _mlp_head(embed, w1a, w1b, b1row):
    n, d = embed.shape
    h = w1a.shape[1]
    br = 400
    assert n % br == 0

    def body(e_ref, wa_ref, wb_ref, b1_ref, p_ref, q_ref):
        e = e_ref[...]
        p_ref[...] = (
            jnp.dot(e, wa_ref[...], preferred_element_type=jnp.float32)
            + b1_ref[...]
        )
        q_ref[...] = jnp.dot(e, wb_ref[...], preferred_element_type=jnp.float32)

    return pl.pallas_call(
        body,
        grid=(n // br,),
        in_specs=[
            pl.BlockSpec((br, d), lambda i: (i, 0)),
            pl.BlockSpec((d, h), lambda i: (0, 0)),
            pl.BlockSpec((d, h), lambda i: (0, 0)),
            pl.BlockSpec((1, h), lambda i: (0, 0)),
        ],
        out_specs=[
            pl.BlockSpec((br, h), lambda i: (i, 0)),
            pl.BlockSpec((br, h), lambda i: (i, 0)),
        ],
        out_shape=[
            jax.ShapeDtypeStruct((n, h), jnp.float32),
            jax.ShapeDtypeStruct((n, h), jnp.float32),
        ],
    )(embed, w1a, w1b, b1row)


def _gather_sum(p, q, edge_index, e):
    n, h = p.shape
    nw = _NCORES * _NSUB
    et = e // nw
    rows = n // _NSUB
    ch = 800
    chunks = []
    off = 0
    while off < et:
        sz = min(ch, et - off)
        chunks.append((off, sz))
        off += sz
    mesh = plsc.VectorSubcoreMesh(core_axis_name="c", subcore_axis_name="s")

    @functools.partial(
        pl.kernel,
        out_type=jax.ShapeDtypeStruct((e, h), jnp.float32),
        mesh=mesh,
        scratch_types=[
            pltpu.VMEM((et,), jnp.int32),
            pltpu.VMEM((et,), jnp.int32),
            pltpu.VMEM((ch, h), jnp.float32),
            pltpu.VMEM((ch, h), jnp.float32),
            pltpu.VMEM_SHARED((n, h), jnp.float32),
            pltpu.VMEM_SHARED((n, h), jnp.float32),
        ],
    )
    def kern(p_hbm, q_hbm, e_hbm, g_hbm, colv, rowv, bufa, bufb, psh, qsh):
        c = lax.axis_index("c")
        s = lax.axis_index("s")
        wid = c * _NSUB + s
        pltpu.sync_copy(p_hbm.at[pl.ds(s * rows, rows), :],
                        bufa.at[pl.ds(0, rows), :])
        pltpu.sync_copy(bufa.at[pl.ds(0, rows), :],
                        psh.at[pl.ds(s * rows, rows), :])
        pltpu.sync_copy(q_hbm.at[pl.ds(s * rows, rows), :],
                        bufb.at[pl.ds(0, rows), :])
        pltpu.sync_copy(bufb.at[pl.ds(0, rows), :],
                        qsh.at[pl.ds(s * rows, rows), :])
        plsc.subcore_barrier()
        base = wid * et
        pltpu.sync_copy(e_hbm.at[0, pl.ds(base, et)], colv)
        pltpu.sync_copy(e_hbm.at[1, pl.ds(base, et)], rowv)
        nv = h // _LANES
        for off, sz in chunks:
            pltpu.sync_copy(psh.at[colv.at[pl.ds(off, sz)]],
                            bufa.at[pl.ds(0, sz), :])
            pltpu.sync_copy(qsh.at[rowv.at[pl.ds(off, sz)]],
                            bufb.at[pl.ds(0, sz), :])

            def add_row(i, _, _nv=nv):
                for j in range(_nv):
                    sl = pl.ds(j * _LANES, _LANES)
                    bufa[i, sl] = bufa[i, sl] + bufb[i, sl]
                return 0

            lax.fori_loop(0, sz, add_row, 0)
            pltpu.sync_copy(bufa.at[pl.ds(0, sz), :],
                            g_hbm.at[pl.ds(base + off, sz), :])

    return kern(p, q, edge_index)


def _edge_score(g, w2row, b2v, e):
    h = g.shape[1]
    br = 3200
    assert e % br == 0

    def body(g_ref, w2_ref, b2_ref, o_ref):
        a = jnp.maximum(g_ref[...], 0.0)
        v = jnp.sum(a * w2_ref[...], axis=1) + b2_ref[0, 0]
        o_ref[...] = jax.nn.sigmoid(v)

    return pl.pallas_call(
        body,
        grid=(e // br,),
        in_specs=[
            pl.BlockSpec((br, h), lambda i: (i, 0)),
            pl.BlockSpec((1, h), lambda i: (0, 0)),
            pl.BlockSpec((1, 1), lambda i: (0, 0)),
        ],
        out_specs=pl.BlockSpec((br,), lambda i: (i,)),
        out_shape=jax.ShapeDtypeStruct((e,), jnp.float32),
    )(g, w2row, b2v)


_HASH_MULS = (-1640531527, -2048144789, -1028477387, 668265263, 374761393)
_TBITS = 19
_TSIZE = 1 << _TBITS


def _dedup_sum(edge_index, values, e, n):
    et = e // _NSUB
    npad = ((et + 127) // 128) * 128
    nvr = npad // _LANES
    nreal = et // _LANES
    mesh = plsc.VectorSubcoreMesh(core_axis_name="c", subcore_axis_name="s")

    @functools.partial(
        pl.kernel,
        out_type=jax.ShapeDtypeStruct((e,), jnp.float32),
        mesh=mesh,
        scratch_types=[
            pltpu.VMEM((npad,), jnp.int32),
            pltpu.VMEM((npad,), jnp.int32),
            pltpu.VMEM((npad,), jnp.int32),
            pltpu.VMEM((npad,), jnp.int32),
            pltpu.VMEM((npad,), jnp.float32),
            pltpu.VMEM((npad,), jnp.int32),
            pltpu.VMEM((npad,), jnp.float32),
            pltpu.VMEM((npad,), jnp.float32),
            pltpu.VMEM((npad,), jnp.float32),
            pltpu.VMEM((npad,), jnp.float32),
            pltpu.VMEM_SHARED((_TSIZE + _NSUB,), jnp.int32),
            pltpu.VMEM_SHARED((_TSIZE + _NSUB,), jnp.float32),
        ],
    )
    def kern(e_hbm, v_hbm, o_hbm, colb, rowb, keyb, hb, valb, gib, gvb,
             addb, outb, zb, ktab, vtab):
        c = lax.axis_index("c")
        s = lax.axis_index("s")

        @pl.when(c == 0)
        def _():
            base = s * et
            pltpu.sync_copy(e_hbm.at[0, pl.ds(base, et)],
                            colb.at[pl.ds(0, et)])
            pltpu.sync_copy(e_hbm.at[1, pl.ds(base, et)],
                            rowb.at[pl.ds(0, et)])
            pltpu.sync_copy(v_hbm.at[pl.ds(base, et)],
                            valb.at[pl.ds(0, et)])
            dump = _TSIZE + s

            def init_body(i, _):
                sl = pl.ds(i * _LANES, _LANES)
                keyb[sl] = colb[sl] * n + rowb[sl]
                outb[sl] = jnp.zeros((_LANES,), jnp.float32)
                zb[sl] = jnp.zeros((_LANES,), jnp.float32)
                return 0

            lax.fori_loop(0, nreal, init_body, 0)
            for i in range(nreal, nvr):
                sl = pl.ds(i * _LANES, _LANES)
                keyb[sl] = jnp.full((_LANES,), -1, jnp.int32)
                valb[sl] = jnp.zeros((_LANES,), jnp.float32)
                outb[sl] = jnp.zeros((_LANES,), jnp.float32)
                zb[sl] = jnp.zeros((_LANES,), jnp.float32)

            for lvl in range(len(_HASH_MULS)):
                mul = jnp.int32(_HASH_MULS[lvl])
                dumpv = jnp.zeros((_LANES,), jnp.int32) + dump

                def hash_body(i, _, _mul=mul, _dumpv=dumpv):
                    sl = pl.ds(i * _LANES, _LANES)
                    k = keyb[sl]
                    hh = lax.shift_right_logical(k * _mul,
                                                 jnp.int32(32 - _TBITS))
                    hb[sl] = jnp.where(k >= 0, hh, _dumpv)
                    return 0

                lax.fori_loop(0, nvr, hash_body, 0)
                pltpu.sync_copy(keyb, ktab.at[hb])
                pltpu.sync_copy(zb, vtab.at[hb])
                plsc.subcore_barrier()
                pltpu.sync_copy(ktab.at[hb], gib)

                def addsrc_body(i, _):
                    sl = pl.ds(i * _LANES, _LANES)
                    k = keyb[sl]
                    w = (gib[sl] == k) & (k >= 0)
                    addb[sl] = jnp.where(w, valb[sl],
                                         jnp.zeros((_LANES,), jnp.float32))
                    return 0

                lax.fori_loop(0, nvr, addsrc_body, 0)
                pltpu.sync_copy(addb, vtab.at[hb], add=True)
                plsc.subcore_barrier()
                pltpu.sync_copy(vtab.at[hb], gvb)

                def retire_body(i, _):
                    sl = pl.ds(i * _LANES, _LANES)
                    k = keyb[sl]
                    w = (gib[sl] == k) & (k >= 0)
                    outb[sl] = jnp.where(w, gvb[sl], outb[sl])
                    keyb[sl] = jnp.where(w, jnp.full((_LANES,), -1, jnp.int32),
                                         k)
                    return 0

                lax.fori_loop(0, nvr, retire_body, 0)
                plsc.subcore_barrier()

            pltpu.sync_copy(outb.at[pl.ds(0, et)], o_hbm.at[pl.ds(base, et)])

    return kern(edge_index, values)


def kernel(embed, edge_index, W1, b1, W2, b2):
    n, d = embed.shape
    e = edge_index.shape[1]
    h = W1.shape[1]
    w1a = W1[:d]
    w1b = W1[d:]
    b1row = b1.reshape(1, h)
    w2row = W2.reshape(1, h)
    b2v = b2.reshape(1, 1)
    p, q = _mlp_head(embed, w1a, w1b, b1row)
    g = _gather_sum(p, q, edge_index, e)
    values = _edge_score(g, w2row, b2v, e)
    return _dedup_sum(edge_index, values, e, n)


# speedup vs baseline: 1.8079x; 1.8079x over previous
import functools

import jax
import jax.numpy as jnp
from jax import lax
from jax.experimental import pallas as pl
from jax.experimental.pallas import tpu as pltpu
from jax.experimental.pallas import tpu_sc as plsc

_NSUB = 16
_NCORES = 2
_LANES = 16


def _mlp_head(embed, w1a, w1b, b1row):
    n, d = embed.shape
    h = w1a.shape[1]
    br = 400
    assert n % br == 0

    def body(e_ref, wa_ref, wb_ref, b1_ref, p_ref, q_ref):
        e = e_ref[...]
        p_ref[...] = (
            jnp.dot(e, wa_ref[...], preferred_element_type=jnp.float32)
            + b1_ref[...]
        )
        q_ref[...] = jnp.dot(e, wb_ref[...], preferred_element_type=jnp.float32)

    return pl.pallas_call(
        body,
        grid=(n // br,),
        in_specs=[
            pl.BlockSpec((br, d), lambda i: (i, 0)),
            pl.BlockSpec((d, h), lambda i: (0, 0)),
            pl.BlockSpec((d, h), lambda i: (0, 0)),
            pl.BlockSpec((1, h), lambda i: (0, 0)),
        ],
        out_specs=[
            pl.BlockSpec((br, h), lambda i: (i, 0)),
            pl.BlockSpec((br, h), lambda i: (i, 0)),
        ],
        out_shape=[
            jax.ShapeDtypeStruct((n, h), jnp.float32),
            jax.ShapeDtypeStruct((n, h), jnp.float32),
        ],
    )(embed, w1a, w1b, b1row)


def _gather_sum(p, q, col, row, e):
    n, h = p.shape
    nw = _NCORES * _NSUB
    et = e // nw
    ch = 800
    chunks = []
    off = 0
    while off < et:
        sz = min(ch, et - off)
        chunks.append((off, sz))
        off += sz
    mesh = plsc.VectorSubcoreMesh(core_axis_name="c", subcore_axis_name="s")

    @functools.partial(
        pl.kernel,
        out_type=jax.ShapeDtypeStruct((e, h), jnp.float32),
        mesh=mesh,
        scratch_types=[
            pltpu.VMEM((et,), jnp.int32),
            pltpu.VMEM((et,), jnp.int32),
            pltpu.VMEM((ch, h), jnp.float32),
            pltpu.VMEM((ch, h), jnp.float32),
        ],
        compiler_params=pltpu.CompilerParams(use_tc_tiling_on_sc=False),
    )
    def kern(p_hbm, q_hbm, col_hbm, row_hbm, g_hbm,
             colv, rowv, bufa, bufb):
        c = lax.axis_index("c")
        s = lax.axis_index("s")
        wid = c * _NSUB + s
        base = wid * et
        pltpu.sync_copy(col_hbm.at[pl.ds(base, et)], colv)
        pltpu.sync_copy(row_hbm.at[pl.ds(base, et)], rowv)
        nv = h // _LANES
        for off, sz in chunks:
            pltpu.sync_copy(p_hbm.at[colv.at[pl.ds(off, sz)]],
                            bufa.at[pl.ds(0, sz), :])
            pltpu.sync_copy(q_hbm.at[rowv.at[pl.ds(off, sz)]],
                            bufb.at[pl.ds(0, sz), :])

            def add_row(i, _, _nv=nv):
                for j in range(_nv):
                    sl = pl.ds(j * _LANES, _LANES)
                    bufa[i, sl] = bufa[i, sl] + bufb[i, sl]
                return 0

            lax.fori_loop(0, sz, add_row, 0)
            pltpu.sync_copy(bufa.at[pl.ds(0, sz), :],
                            g_hbm.at[pl.ds(base + off, sz), :])

    return kern(p, q, col, row)


def _edge_score(g, w2row, b2v, e):
    h = g.shape[1]
    br = 2048

    def body(g_ref, w2_ref, b2_ref, o_ref):
        a = jnp.maximum(g_ref[...], 0.0)
        v = jnp.sum(a * w2_ref[...], axis=1) + b2_ref[0, 0]
        o_ref[...] = jax.nn.sigmoid(v)

    return pl.pallas_call(
        body,
        grid=(pl.cdiv(e, br),),
        in_specs=[
            pl.BlockSpec((br, h), lambda i: (i, 0)),
            pl.BlockSpec((1, h), lambda i: (0, 0)),
            pl.BlockSpec((1, 1), lambda i: (0, 0)),
        ],
        out_specs=pl.BlockSpec((br,), lambda i: (i,)),
        out_shape=jax.ShapeDtypeStruct((e,), jnp.float32),
    )(g, w2row, b2v)


_HASH_MULS = (-1640531527, -2048144789, -1028477387, 668265263,
              374761393, -1700995253, 1181783497)
_TBITS = 17
_TSIZE = 1 << _TBITS


def _dedup_sum(col, row, values, e, n):
    et = e // _NSUB
    npad = ((et + 127) // 128) * 128
    nvr = npad // _LANES
    nreal = et // _LANES
    mesh = plsc.VectorSubcoreMesh(core_axis_name="c", subcore_axis_name="s")

    @functools.partial(
        pl.kernel,
        out_type=jax.ShapeDtypeStruct((e,), jnp.float32),
        mesh=mesh,
        scratch_types=[
            pltpu.VMEM((npad,), jnp.int32),
            pltpu.VMEM((npad,), jnp.int32),
            pltpu.VMEM((npad,), jnp.int32),
            pltpu.VMEM((npad,), jnp.int32),
            pltpu.VMEM((npad,), jnp.float32),
            pltpu.VMEM((npad,), jnp.int32),
            pltpu.VMEM((npad,), jnp.float32),
            pltpu.VMEM((npad,), jnp.float32),
            pltpu.VMEM((npad,), jnp.float32),
            pltpu.VMEM((npad,), jnp.float32),
            pltpu.VMEM_SHARED((_TSIZE + _NSUB,), jnp.int32),
            pltpu.VMEM_SHARED((_TSIZE + _NSUB,), jnp.float32),
        ],
        compiler_params=pltpu.CompilerParams(use_tc_tiling_on_sc=False),
    )
    def kern(col_hbm, row_hbm, v_hbm, o_hbm, colb, rowb, keyb, hb, valb,
             gib, gvb, addb, outb, zb, ktab, vtab):
        c = lax.axis_index("c")
        s = lax.axis_index("s")

        @pl.when(c == 0)
        def _():
            base = s * et
            pltpu.sync_copy(col_hbm.at[pl.ds(base, et)],
                            colb.at[pl.ds(0, et)])
            pltpu.sync_copy(row_hbm.at[pl.ds(base, et)],
                            rowb.at[pl.ds(0, et)])
            pltpu.sync_copy(v_hbm.at[pl.ds(base, et)],
                            valb.at[pl.ds(0, et)])
            dump = _TSIZE + s

            def init_body(i, _):
                sl = pl.ds(i * _LANES, _LANES)
                keyb[sl] = colb[sl] * n + rowb[sl]
                outb[sl] = jnp.zeros((_LANES,), jnp.float32)
                zb[sl] = jnp.zeros((_LANES,), jnp.float32)
                return 0

            lax.fori_loop(0, nreal, init_body, 0)
            for i in range(nreal, nvr):
                sl = pl.ds(i * _LANES, _LANES)
                keyb[sl] = jnp.full((_LANES,), -1, jnp.int32)
                valb[sl] = jnp.zeros((_LANES,), jnp.float32)
                outb[sl] = jnp.zeros((_LANES,), jnp.float32)
                zb[sl] = jnp.zeros((_LANES,), jnp.float32)

            for lvl in range(len(_HASH_MULS)):
                mul = jnp.int32(_HASH_MULS[lvl])
                dumpv = jnp.zeros((_LANES,), jnp.int32) + dump

                def hash_body(i, _, _mul=mul, _dumpv=dumpv):
                    sl = pl.ds(i * _LANES, _LANES)
                    k = keyb[sl]
                    hh = lax.shift_right_logical(k * _mul,
                                                 jnp.int32(32 - _TBITS))
                    hb[sl] = jnp.where(k >= 0, hh, _dumpv)
                    return 0

                lax.fori_loop(0, nvr, hash_body, 0)
                pltpu.sync_copy(keyb, ktab.at[hb])
                pltpu.sync_copy(zb, vtab.at[hb])
                plsc.subcore_barrier()
                pltpu.sync_copy(ktab.at[hb], gib)

                def addsrc_body(i, _):
                    sl = pl.ds(i * _LANES, _LANES)
                    k = keyb[sl]
                    w = (gib[sl] == k) & (k >= 0)
                    addb[sl] = jnp.where(w, valb[sl],
                                         jnp.zeros((_LANES,), jnp.float32))
                    return 0

                lax.fori_loop(0, nvr, addsrc_body, 0)
                pltpu.sync_copy(addb, vtab.at[hb], add=True)
                plsc.subcore_barrier()
                pltpu.sync_copy(vtab.at[hb], gvb)

                def retire_body(i, _):
                    sl = pl.ds(i * _LANES, _LANES)
                    k = keyb[sl]
                    w = (gib[sl] == k) & (k >= 0)
                    outb[sl] = jnp.where(w, gvb[sl], outb[sl])
                    keyb[sl] = jnp.where(w, jnp.full((_LANES,), -1, jnp.int32),
                                         k)
                    return 0

                lax.fori_loop(0, nvr, retire_body, 0)
                plsc.subcore_barrier()

            pltpu.sync_copy(outb.at[pl.ds(0, et)], o_hbm.at[pl.ds(base, et)])

    return kern(col, row, values)


def kernel(embed, edge_index, W1, b1, W2, b2):
    n, d = embed.shape
    e = edge_index.shape[1]
    h = W1.shape[1]
    w1a = W1[:d]
    w1b = W1[d:]
    b1row = b1.reshape(1, h)
    w2row = W2.reshape(1, h)
    b2v = b2.reshape(1, 1)
    col = edge_index[0]
    row = edge_index[1]
    p, q = _mlp_head(embed, w1a, w1b, b1row)
    g = _gather_sum(p, q, col, row, e)
    values = _edge_score(g, w2row, b2v, e)
    return _dedup_sum(col, row, values, e, n)


# speedup vs baseline: 1.8723x; 1.0356x over previous
import functools

import jax
import jax.numpy as jnp
from jax import lax
from jax.experimental import pallas as pl
from jax.experimental.pallas import tpu as pltpu
from jax.experimental.pallas import tpu_sc as plsc

_NSUB = 16
_NCORES = 2
_LANES = 16


def _mlp_head(embed, w1a, w1b, b1row):
    n, d = embed.shape
    h = w1a.shape[1]
    br = 400
    assert n % br == 0

    def body(e_ref, wa_ref, wb_ref, b1_ref, p_ref, q_ref):
        e = e_ref[...]
        p_ref[...] = (
            jnp.dot(e, wa_ref[...], preferred_element_type=jnp.float32)
            + b1_ref[...]
        )
        q_ref[...] = jnp.dot(e, wb_ref[...], preferred_element_type=jnp.float32)

    return pl.pallas_call(
        body,
        grid=(n // br,),
        in_specs=[
            pl.BlockSpec((br, d), lambda i: (i, 0)),
            pl.BlockSpec((d, h), lambda i: (0, 0)),
            pl.BlockSpec((d, h), lambda i: (0, 0)),
            pl.BlockSpec((1, h), lambda i: (0, 0)),
        ],
        out_specs=[
            pl.BlockSpec((br, h), lambda i: (i, 0)),
            pl.BlockSpec((br, h), lambda i: (i, 0)),
        ],
        out_shape=[
            jax.ShapeDtypeStruct((n, h), jnp.float32),
            jax.ShapeDtypeStruct((n, h), jnp.float32),
        ],
    )(embed, w1a, w1b, b1row)


def _gather_sum(p, q, col, row, e):
    n, h = p.shape
    nw = _NCORES * _NSUB
    et = e // nw
    ch = 800
    chunks = []
    off = 0
    while off < et:
        sz = min(ch, et - off)
        chunks.append((off, sz))
        off += sz
    mesh = plsc.VectorSubcoreMesh(core_axis_name="c", subcore_axis_name="s")

    @functools.partial(
        pl.kernel,
        out_type=jax.ShapeDtypeStruct((e, h), jnp.float32),
        mesh=mesh,
        scratch_types=[
            pltpu.VMEM((et,), jnp.int32),
            pltpu.VMEM((et,), jnp.int32),
            pltpu.VMEM((ch, h), jnp.float32),
            pltpu.VMEM((ch, h), jnp.float32),
        ],
        compiler_params=pltpu.CompilerParams(use_tc_tiling_on_sc=False),
    )
    def kern(p_hbm, q_hbm, col_hbm, row_hbm, g_hbm,
             colv, rowv, bufa, bufb):
        c = lax.axis_index("c")
        s = lax.axis_index("s")
        wid = c * _NSUB + s
        base = wid * et
        pltpu.sync_copy(col_hbm.at[pl.ds(base, et)], colv)
        pltpu.sync_copy(row_hbm.at[pl.ds(base, et)], rowv)
        nv = h // _LANES
        for off, sz in chunks:
            pltpu.sync_copy(p_hbm.at[colv.at[pl.ds(off, sz)]],
                            bufa.at[pl.ds(0, sz), :])
            pltpu.sync_copy(q_hbm.at[rowv.at[pl.ds(off, sz)]],
                            bufb.at[pl.ds(0, sz), :])

            def add_row(i, _, _nv=nv):
                for j in range(_nv):
                    sl = pl.ds(j * _LANES, _LANES)
                    bufa[i, sl] = bufa[i, sl] + bufb[i, sl]
                return 0

            lax.fori_loop(0, sz, add_row, 0)
            pltpu.sync_copy(bufa.at[pl.ds(0, sz), :],
                            g_hbm.at[pl.ds(base + off, sz), :])

    return kern(p, q, col, row)


def _edge_score(g, w2row, b2v, e):
    h = g.shape[1]
    br = 2048

    def body(g_ref, w2_ref, b2_ref, o_ref):
        a = jnp.maximum(g_ref[...], 0.0)
        v = jnp.sum(a * w2_ref[...], axis=1) + b2_ref[0, 0]
        o_ref[...] = jax.nn.sigmoid(v)

    return pl.pallas_call(
        body,
        grid=(pl.cdiv(e, br),),
        in_specs=[
            pl.BlockSpec((br, h), lambda i: (i, 0)),
            pl.BlockSpec((1, h), lambda i: (0, 0)),
            pl.BlockSpec((1, 1), lambda i: (0, 0)),
        ],
        out_specs=pl.BlockSpec((br,), lambda i: (i,)),
        out_shape=jax.ShapeDtypeStruct((e,), jnp.float32),
    )(g, w2row, b2v)


_HASH_MULS = (-1640531527, -2048144789, -1028477387, 668265263,
              374761393, -1700995253, 1181783497)
_TBITS = 17
_TSIZE = 1 << _TBITS


def _dedup_sum(col, row, values, e, n):
    et = e // _NSUB
    npad = ((et + 127) // 128) * 128
    nvr = npad // _LANES
    nreal = et // _LANES
    unroll = 4
    assert nvr % unroll == 0
    mesh = plsc.VectorSubcoreMesh(core_axis_name="c", subcore_axis_name="s")

    @functools.partial(
        pl.kernel,
        out_type=jax.ShapeDtypeStruct((e,), jnp.float32),
        mesh=mesh,
        scratch_types=[
            pltpu.VMEM((npad,), jnp.int32),
            pltpu.VMEM((npad,), jnp.int32),
            pltpu.VMEM((npad,), jnp.int32),
            pltpu.VMEM((npad,), jnp.int32),
            pltpu.VMEM((npad,), jnp.float32),
            pltpu.VMEM((npad,), jnp.int32),
            pltpu.VMEM((npad,), jnp.float32),
            pltpu.VMEM((npad,), jnp.float32),
            pltpu.VMEM((npad,), jnp.float32),
            pltpu.VMEM((npad,), jnp.float32),
            pltpu.VMEM((_LANES,), jnp.int32),
            pltpu.VMEM((_NSUB * _LANES,), jnp.int32),
            pltpu.VMEM((_LANES,), jnp.int32),
            pltpu.VMEM_SHARED((_TSIZE + _NSUB,), jnp.int32),
            pltpu.VMEM_SHARED((_TSIZE + _NSUB,), jnp.float32),
            pltpu.VMEM_SHARED((_NSUB * _LANES,), jnp.int32),
        ],
        compiler_params=pltpu.CompilerParams(use_tc_tiling_on_sc=False),
    )
    def kern(col_hbm, row_hbm, v_hbm, o_hbm, colb, rowb, keyb, hb, valb,
             gib, gvb, addb, outb, zb, cntv, cntall, totv, ktab, vtab,
             cntsh):
        c = lax.axis_index("c")
        s = lax.axis_index("s")

        @pl.when(c == 0)
        def _():
            base = s * et
            pltpu.sync_copy(col_hbm.at[pl.ds(base, et)],
                            colb.at[pl.ds(0, et)])
            pltpu.sync_copy(row_hbm.at[pl.ds(base, et)],
                            rowb.at[pl.ds(0, et)])
            pltpu.sync_copy(v_hbm.at[pl.ds(base, et)],
                            valb.at[pl.ds(0, et)])
            dump = _TSIZE + s
            dumpv = jnp.zeros((_LANES,), jnp.int32) + dump
            zv = jnp.zeros((_LANES,), jnp.float32)
            neg1 = jnp.full((_LANES,), -1, jnp.int32)
            mul0 = jnp.int32(_HASH_MULS[0])
            shift = jnp.int32(32 - _TBITS)

            def init_body(i, _):
                for u in range(unroll):
                    sl = pl.ds((i * unroll + u) * _LANES, _LANES)
                    k = colb[sl] * n + rowb[sl]
                    keyb[sl] = k
                    hb[sl] = lax.shift_right_logical(k * mul0, shift)
                    outb[sl] = zv
                    zb[sl] = zv
                return 0

            lax.fori_loop(0, nreal // unroll, init_body, 0)
            for i in range((nreal // unroll) * unroll, nreal):
                sl = pl.ds(i * _LANES, _LANES)
                k = colb[sl] * n + rowb[sl]
                keyb[sl] = k
                hb[sl] = lax.shift_right_logical(k * mul0, shift)
                outb[sl] = zv
                zb[sl] = zv
            for i in range(nreal, nvr):
                sl = pl.ds(i * _LANES, _LANES)
                keyb[sl] = neg1
                hb[sl] = dumpv
                valb[sl] = zv
                outb[sl] = zv
                zb[sl] = zv

            for lvl in range(len(_HASH_MULS)):
                last = lvl == len(_HASH_MULS) - 1
                muln = jnp.int32(_HASH_MULS[min(lvl + 1, len(_HASH_MULS) - 1)])

                def level_body(_muln=muln, _last=last):
                    pltpu.sync_copy(keyb, ktab.at[hb])
                    pltpu.sync_copy(zb, vtab.at[hb])
                    plsc.subcore_barrier()
                    pltpu.sync_copy(ktab.at[hb], gib)

                    def addsrc_body(i, _):
                        for u in range(unroll):
                            sl = pl.ds((i * unroll + u) * _LANES, _LANES)
                            k = keyb[sl]
                            w = (gib[sl] == k) & (k >= 0)
                            addb[sl] = jnp.where(w, valb[sl], zv)
                        return 0

                    lax.fori_loop(0, nvr // unroll, addsrc_body, 0)
                    pltpu.sync_copy(addb, vtab.at[hb], add=True)
                    plsc.subcore_barrier()
                    pltpu.sync_copy(vtab.at[hb], gvb)

                    def retire_body(i, _):
                        for u in range(unroll):
                            sl = pl.ds((i * unroll + u) * _LANES, _LANES)
                            k = keyb[sl]
                            w = (gib[sl] == k) & (k >= 0)
                            outb[sl] = jnp.where(w, gvb[sl], outb[sl])
                            k2 = jnp.where(w, neg1, k)
                            keyb[sl] = k2
                            if not _last:
                                hh = lax.shift_right_logical(k2 * _muln,
                                                             shift)
                                hb[sl] = jnp.where(k2 >= 0, hh, dumpv)
                        return 0

                    lax.fori_loop(0, nvr // unroll, retire_body, 0)
                    plsc.subcore_barrier()

                level_body()

            pltpu.sync_copy(outb.at[pl.ds(0, et)], o_hbm.at[pl.ds(base, et)])

    return kern(col, row, values)


def kernel(embed, edge_index, W1, b1, W2, b2):
    n, d = embed.shape
    e = edge_index.shape[1]
    h = W1.shape[1]
    w1a = W1[:d]
    w1b = W1[d:]
    b1row = b1.reshape(1, h)
    w2row = W2.reshape(1, h)
    b2v = b2.reshape(1, 1)
    col = edge_index[0]
    row = edge_index[1]
    p, q = _mlp_head(embed, w1a, w1b, b1row)
    g = _gather_sum(p, q, col, row, e)
    values = _edge_score(g, w2row, b2v, e)
    return _dedup_sum(col, row, values, e, n)


# speedup vs baseline: 2.5017x; 1.3362x over previous
import functools

import jax
import jax.numpy as jnp
from jax import lax
from jax.experimental import pallas as pl
from jax.experimental.pallas import tpu as pltpu
from jax.experimental.pallas import tpu_sc as plsc

_NSUB = 16
_NCORES = 2
_LANES = 16


def _mlp_head(embed, w1a, w1b, b1row):
    n, d = embed.shape
    h = w1a.shape[1]
    br = 400
    assert n % br == 0

    def body(e_ref, wa_ref, wb_ref, b1_ref, p_ref, q_ref):
        e = e_ref[...]
        p_ref[...] = (
            jnp.dot(e, wa_ref[...], preferred_element_type=jnp.float32)
            + b1_ref[...]
        )
        q_ref[...] = jnp.dot(e, wb_ref[...], preferred_element_type=jnp.float32)

    return pl.pallas_call(
        body,
        grid=(n // br,),
        in_specs=[
            pl.BlockSpec((br, d), lambda i: (i, 0)),
            pl.BlockSpec((d, h), lambda i: (0, 0)),
            pl.BlockSpec((d, h), lambda i: (0, 0)),
            pl.BlockSpec((1, h), lambda i: (0, 0)),
        ],
        out_specs=[
            pl.BlockSpec((br, h), lambda i: (i, 0)),
            pl.BlockSpec((br, h), lambda i: (i, 0)),
        ],
        out_shape=[
            jax.ShapeDtypeStruct((n, h), jnp.float32),
            jax.ShapeDtypeStruct((n, h), jnp.float32),
        ],
    )(embed, w1a, w1b, b1row)


def _gather_sum(p, q, col, row, e):
    n, h = p.shape
    nw = _NCORES * _NSUB
    et = e // nw
    ch = 800
    chunks = []
    off = 0
    while off < et:
        sz = min(ch, et - off)
        chunks.append((off, sz))
        off += sz
    mesh = plsc.VectorSubcoreMesh(core_axis_name="c", subcore_axis_name="s")

    @functools.partial(
        pl.kernel,
        out_type=jax.ShapeDtypeStruct((e, h), jnp.float32),
        mesh=mesh,
        scratch_types=[
            pltpu.VMEM((et,), jnp.int32),
            pltpu.VMEM((et,), jnp.int32),
            pltpu.VMEM((ch, h), jnp.float32),
            pltpu.VMEM((ch, h), jnp.float32),
        ],
        compiler_params=pltpu.CompilerParams(use_tc_tiling_on_sc=False),
    )
    def kern(p_hbm, q_hbm, col_hbm, row_hbm, g_hbm,
             colv, rowv, bufa, bufb):
        c = lax.axis_index("c")
        s = lax.axis_index("s")
        wid = c * _NSUB + s
        base = wid * et
        pltpu.sync_copy(col_hbm.at[pl.ds(base, et)], colv)
        pltpu.sync_copy(row_hbm.at[pl.ds(base, et)], rowv)
        nv = h // _LANES
        for off, sz in chunks:
            pltpu.sync_copy(p_hbm.at[colv.at[pl.ds(off, sz)]],
                            bufa.at[pl.ds(0, sz), :])
            pltpu.sync_copy(q_hbm.at[rowv.at[pl.ds(off, sz)]],
                            bufb.at[pl.ds(0, sz), :])

            def add_row(i, _, _nv=nv):
                for j in range(_nv):
                    sl = pl.ds(j * _LANES, _LANES)
                    bufa[i, sl] = bufa[i, sl] + bufb[i, sl]
                return 0

            lax.fori_loop(0, sz, add_row, 0)
            pltpu.sync_copy(bufa.at[pl.ds(0, sz), :],
                            g_hbm.at[pl.ds(base + off, sz), :])

    return kern(p, q, col, row)


def _edge_score(g, w2row, b2v, e):
    h = g.shape[1]
    br = 2048

    def body(g_ref, w2_ref, b2_ref, o_ref):
        a = jnp.maximum(g_ref[...], 0.0)
        v = jnp.sum(a * w2_ref[...], axis=1) + b2_ref[0, 0]
        o_ref[...] = jax.nn.sigmoid(v)

    return pl.pallas_call(
        body,
        grid=(pl.cdiv(e, br),),
        in_specs=[
            pl.BlockSpec((br, h), lambda i: (i, 0)),
            pl.BlockSpec((1, h), lambda i: (0, 0)),
            pl.BlockSpec((1, 1), lambda i: (0, 0)),
        ],
        out_specs=pl.BlockSpec((br,), lambda i: (i,)),
        out_shape=jax.ShapeDtypeStruct((e,), jnp.float32),
    )(g, w2row, b2v)


_HASH_MULS = (-1640531527, -2048144789, -1028477387, 668265263,
              374761393, -1700995253, 1181783497)
_TBITS = 17
_TSIZE = 1 << _TBITS


def _dedup_sum(col, row, values, e, n):
    et = e // _NSUB
    npad = ((et + 127) // 128) * 128
    nvr = npad // _LANES
    nreal = et // _LANES
    unroll = 4
    assert nvr % unroll == 0
    mesh = plsc.VectorSubcoreMesh(core_axis_name="c", subcore_axis_name="s")

    @functools.partial(
        pl.kernel,
        out_type=jax.ShapeDtypeStruct((e,), jnp.float32),
        mesh=mesh,
        scratch_types=[
            pltpu.VMEM((npad,), jnp.int32),
            pltpu.VMEM((npad,), jnp.int32),
            pltpu.VMEM((npad,), jnp.int32),
            pltpu.VMEM((npad,), jnp.int32),
            pltpu.VMEM((npad,), jnp.float32),
            pltpu.VMEM((npad,), jnp.int32),
            pltpu.VMEM((npad,), jnp.float32),
            pltpu.VMEM((npad,), jnp.float32),
            pltpu.VMEM((npad,), jnp.float32),
            pltpu.VMEM((npad,), jnp.float32),
            pltpu.VMEM((_LANES,), jnp.int32),
            pltpu.VMEM((_NSUB * _LANES,), jnp.int32),
            pltpu.VMEM((_LANES,), jnp.int32),
            pltpu.VMEM_SHARED((_TSIZE + _NSUB,), jnp.int32),
            pltpu.VMEM_SHARED((_TSIZE + _NSUB,), jnp.float32),
            pltpu.VMEM_SHARED((_NSUB * _LANES,), jnp.int32),
        ],
        compiler_params=pltpu.CompilerParams(use_tc_tiling_on_sc=False),
    )
    def kern(col_hbm, row_hbm, v_hbm, o_hbm, colb, rowb, keyb, hb, valb,
             gib, gvb, addb, outb, zb, cntv, cntall, totv, ktab, vtab,
             cntsh):
        c = lax.axis_index("c")
        s = lax.axis_index("s")

        @pl.when(c == 0)
        def _():
            base = s * et
            pltpu.sync_copy(col_hbm.at[pl.ds(base, et)],
                            colb.at[pl.ds(0, et)])
            pltpu.sync_copy(row_hbm.at[pl.ds(base, et)],
                            rowb.at[pl.ds(0, et)])
            pltpu.sync_copy(v_hbm.at[pl.ds(base, et)],
                            valb.at[pl.ds(0, et)])
            dump = _TSIZE + s
            dumpv = jnp.zeros((_LANES,), jnp.int32) + dump
            zv = jnp.zeros((_LANES,), jnp.float32)
            neg1 = jnp.full((_LANES,), -1, jnp.int32)
            mul0 = jnp.int32(_HASH_MULS[0])
            shift = jnp.int32(32 - _TBITS)

            def init_body(i, _):
                for u in range(unroll):
                    sl = pl.ds((i * unroll + u) * _LANES, _LANES)
                    k = colb[sl] * n + rowb[sl]
                    keyb[sl] = k
                    hb[sl] = lax.shift_right_logical(k * mul0, shift)
                    outb[sl] = zv
                    zb[sl] = zv
                return 0

            lax.fori_loop(0, nreal // unroll, init_body, 0)
            for i in range((nreal // unroll) * unroll, nreal):
                sl = pl.ds(i * _LANES, _LANES)
                k = colb[sl] * n + rowb[sl]
                keyb[sl] = k
                hb[sl] = lax.shift_right_logical(k * mul0, shift)
                outb[sl] = zv
                zb[sl] = zv
            for i in range(nreal, nvr):
                sl = pl.ds(i * _LANES, _LANES)
                keyb[sl] = neg1
                hb[sl] = dumpv
                valb[sl] = zv
                outb[sl] = zv
                zb[sl] = zv

            for lvl in range(len(_HASH_MULS)):
                last = lvl == len(_HASH_MULS) - 1
                muln = jnp.int32(_HASH_MULS[min(lvl + 1, len(_HASH_MULS) - 1)])

                def level_body(_muln=muln, _last=last):
                    pltpu.sync_copy(keyb, ktab.at[hb])
                    pltpu.sync_copy(zb, vtab.at[hb])
                    plsc.subcore_barrier()
                    pltpu.sync_copy(ktab.at[hb], gib)

                    def addsrc_body(i, _):
                        for u in range(unroll):
                            sl = pl.ds((i * unroll + u) * _LANES, _LANES)
                            k = keyb[sl]
                            w = (gib[sl] == k) & (k >= 0)
                            addb[sl] = jnp.where(w, valb[sl], zv)
                        return 0

                    lax.fori_loop(0, nvr // unroll, addsrc_body, 0)
                    pltpu.sync_copy(addb, vtab.at[hb], add=True)
                    plsc.subcore_barrier()
                    pltpu.sync_copy(vtab.at[hb], gvb)

                    one16 = jnp.full((_LANES,), 1, jnp.int32)
                    zero16 = jnp.zeros((_LANES,), jnp.int32)

                    def retire_body(i, cnt):
                        for u in range(unroll):
                            sl = pl.ds((i * unroll + u) * _LANES, _LANES)
                            k = keyb[sl]
                            w = (gib[sl] == k) & (k >= 0)
                            outb[sl] = jnp.where(w, gvb[sl], outb[sl])
                            k2 = jnp.where(w, neg1, k)
                            keyb[sl] = k2
                            if not _last:
                                hh = lax.shift_right_logical(k2 * _muln,
                                                             shift)
                                hb[sl] = jnp.where(k2 >= 0, hh, dumpv)
                                cnt = cnt + jnp.where(k2 >= 0, one16, zero16)
                        return cnt

                    cnt = lax.fori_loop(0, nvr // unroll, retire_body, zero16)
                    if not _last:
                        cntv[pl.ds(0, _LANES)] = cnt
                    plsc.subcore_barrier()

                if lvl == 0:
                    level_body()
                else:
                    pltpu.sync_copy(cntv, cntsh.at[pl.ds(s * _LANES, _LANES)])
                    plsc.subcore_barrier()
                    pltpu.sync_copy(cntsh, cntall)
                    acc = jnp.zeros((_LANES,), jnp.int32)
                    for i in range(_NSUB):
                        acc = acc + cntall[pl.ds(i * _LANES, _LANES)]
                    totv[pl.ds(0, _LANES)] = acc
                    av = totv[pl.ds(0, _LANES)]
                    t = av[0]
                    for i in range(1, _LANES):
                        t = t + av[i]
                    pl.when(t > 0)(level_body)

            pltpu.sync_copy(outb.at[pl.ds(0, et)], o_hbm.at[pl.ds(base, et)])

    return kern(col, row, values)


def kernel(embed, edge_index, W1, b1, W2, b2):
    n, d = embed.shape
    e = edge_index.shape[1]
    h = W1.shape[1]
    w1a = W1[:d]
    w1b = W1[d:]
    b1row = b1.reshape(1, h)
    w2row = W2.reshape(1, h)
    b2v = b2.reshape(1, 1)
    col = edge_index[0]
    row = edge_index[1]
    p, q = _mlp_head(embed, w1a, w1b, b1row)
    g = _gather_sum(p, q, col, row, e)
    values = _edge_score(g, w2row, b2v, e)
    return _dedup_sum(col, row, values, e, n)


# speedup vs baseline: 2.8072x; 1.1221x over previous
import functools

import jax
import jax.numpy as jnp
from jax import lax
from jax.experimental import pallas as pl
from jax.experimental.pallas import tpu as pltpu
from jax.experimental.pallas import tpu_sc as plsc

_NSUB = 16
_NCORES = 2
_LANES = 16


def _mlp_head(embed, w1a, w1b, b1row):
    n, d = embed.shape
    h = w1a.shape[1]
    br = 400
    assert n % br == 0

    def body(e_ref, wa_ref, wb_ref, b1_ref, p_ref, q_ref):
        e = e_ref[...]
        p_ref[...] = (
            jnp.dot(e, wa_ref[...], preferred_element_type=jnp.float32)
            + b1_ref[...]
        )
        q_ref[...] = jnp.dot(e, wb_ref[...], preferred_element_type=jnp.float32)

    return pl.pallas_call(
        body,
        grid=(n // br,),
        in_specs=[
            pl.BlockSpec((br, d), lambda i: (i, 0)),
            pl.BlockSpec((d, h), lambda i: (0, 0)),
            pl.BlockSpec((d, h), lambda i: (0, 0)),
            pl.BlockSpec((1, h), lambda i: (0, 0)),
        ],
        out_specs=[
            pl.BlockSpec((br, h), lambda i: (i, 0)),
            pl.BlockSpec((br, h), lambda i: (i, 0)),
        ],
        out_shape=[
            jax.ShapeDtypeStruct((n, h), jnp.float32),
            jax.ShapeDtypeStruct((n, h), jnp.float32),
        ],
    )(embed, w1a, w1b, b1row)


def _gather_sum(p, q, col, row, e):
    n, h = p.shape
    nw = _NCORES * _NSUB
    et = e // nw
    ch = 800
    chunks = []
    off = 0
    while off < et:
        sz = min(ch, et - off)
        chunks.append((off, sz))
        off += sz
    mesh = plsc.VectorSubcoreMesh(core_axis_name="c", subcore_axis_name="s")

    @functools.partial(
        pl.kernel,
        out_type=jax.ShapeDtypeStruct((e, h), jnp.float32),
        mesh=mesh,
        scratch_types=[
            pltpu.VMEM((et,), jnp.int32),
            pltpu.VMEM((et,), jnp.int32),
            pltpu.VMEM((ch, h), jnp.float32),
            pltpu.VMEM((ch, h), jnp.float32),
        ],
        compiler_params=pltpu.CompilerParams(use_tc_tiling_on_sc=False),
    )
    def kern(p_hbm, q_hbm, col_hbm, row_hbm, g_hbm,
             colv, rowv, bufa, bufb):
        c = lax.axis_index("c")
        s = lax.axis_index("s")
        wid = c * _NSUB + s
        base = wid * et
        pltpu.sync_copy(col_hbm.at[pl.ds(base, et)], colv)
        pltpu.sync_copy(row_hbm.at[pl.ds(base, et)], rowv)
        nv = h // _LANES
        for off, sz in chunks:
            pltpu.sync_copy(p_hbm.at[colv.at[pl.ds(off, sz)]],
                            bufa.at[pl.ds(0, sz), :])
            pltpu.sync_copy(q_hbm.at[rowv.at[pl.ds(off, sz)]],
                            bufb.at[pl.ds(0, sz), :])

            def add_row(i, _, _nv=nv):
                for j in range(_nv):
                    sl = pl.ds(j * _LANES, _LANES)
                    bufa[i, sl] = bufa[i, sl] + bufb[i, sl]
                return 0

            lax.fori_loop(0, sz, add_row, 0)
            pltpu.sync_copy(bufa.at[pl.ds(0, sz), :],
                            g_hbm.at[pl.ds(base + off, sz), :])

    return kern(p, q, col, row)


def _edge_score(g, w2row, b2v, e):
    h = g.shape[1]
    br = 2048

    def body(g_ref, w2_ref, b2_ref, o_ref):
        a = jnp.maximum(g_ref[...], 0.0)
        v = jnp.sum(a * w2_ref[...], axis=1) + b2_ref[0, 0]
        o_ref[...] = jax.nn.sigmoid(v)

    return pl.pallas_call(
        body,
        grid=(pl.cdiv(e, br),),
        in_specs=[
            pl.BlockSpec((br, h), lambda i: (i, 0)),
            pl.BlockSpec((1, h), lambda i: (0, 0)),
            pl.BlockSpec((1, 1), lambda i: (0, 0)),
        ],
        out_specs=pl.BlockSpec((br,), lambda i: (i,)),
        out_shape=jax.ShapeDtypeStruct((e,), jnp.float32),
    )(g, w2row, b2v)


_HASH_MULS = (-1640531527, -2048144789, -1028477387, 668265263,
              374761393, -1700995253, 1181783497)
_TBITS = 17
_TSIZE = 1 << _TBITS


def _dedup_sum(col, row, values, e, n):
    et = e // _NSUB
    npad = ((et + 127) // 128) * 128
    nvr = npad // _LANES
    nreal = et // _LANES
    unroll = 4
    assert nvr % unroll == 0
    mesh = plsc.VectorSubcoreMesh(core_axis_name="c", subcore_axis_name="s")

    @functools.partial(
        pl.kernel,
        out_type=jax.ShapeDtypeStruct((e,), jnp.float32),
        mesh=mesh,
        scratch_types=[
            pltpu.VMEM((npad,), jnp.int32),
            pltpu.VMEM((npad,), jnp.int32),
            pltpu.VMEM((npad,), jnp.int32),
            pltpu.VMEM((npad,), jnp.int32),
            pltpu.VMEM((npad,), jnp.float32),
            pltpu.VMEM((npad,), jnp.int32),
            pltpu.VMEM((npad,), jnp.float32),
            pltpu.VMEM((npad,), jnp.float32),
            pltpu.VMEM((npad,), jnp.float32),
            pltpu.VMEM((npad,), jnp.float32),
            pltpu.VMEM((_LANES,), jnp.int32),
            pltpu.VMEM((_NSUB * _LANES,), jnp.int32),
            pltpu.VMEM((_LANES,), jnp.int32),
            pltpu.VMEM_SHARED((_TSIZE + _NSUB,), jnp.int32),
            pltpu.VMEM_SHARED((_TSIZE + _NSUB,), jnp.float32),
            pltpu.VMEM_SHARED((_NSUB * _LANES,), jnp.int32),
        ],
        compiler_params=pltpu.CompilerParams(use_tc_tiling_on_sc=False),
    )
    def kern(col_hbm, row_hbm, v_hbm, o_hbm, colb, rowb, keyb, hb, valb,
             gib, gvb, addb, outb, zb, cntv, cntall, totv, ktab, vtab,
             cntsh):
        c = lax.axis_index("c")
        s = lax.axis_index("s")

        @pl.when(c == 0)
        def _():
            base = s * et
            pltpu.sync_copy(col_hbm.at[pl.ds(base, et)],
                            colb.at[pl.ds(0, et)])
            pltpu.sync_copy(row_hbm.at[pl.ds(base, et)],
                            rowb.at[pl.ds(0, et)])
            pltpu.sync_copy(v_hbm.at[pl.ds(base, et)],
                            valb.at[pl.ds(0, et)])
            dump = _TSIZE + s
            dumpv = jnp.zeros((_LANES,), jnp.int32) + dump
            zv = jnp.zeros((_LANES,), jnp.float32)
            neg1 = jnp.full((_LANES,), -1, jnp.int32)
            mul0 = jnp.int32(_HASH_MULS[0])
            shift = jnp.int32(32 - _TBITS)

            def init_body(i, _):
                for u in range(unroll):
                    sl = pl.ds((i * unroll + u) * _LANES, _LANES)
                    k = colb[sl] * n + rowb[sl]
                    keyb[sl] = k
                    hb[sl] = lax.shift_right_logical(k * mul0, shift)
                    outb[sl] = zv
                    zb[sl] = zv
                return 0

            lax.fori_loop(0, nreal // unroll, init_body, 0)
            for i in range((nreal // unroll) * unroll, nreal):
                sl = pl.ds(i * _LANES, _LANES)
                k = colb[sl] * n + rowb[sl]
                keyb[sl] = k
                hb[sl] = lax.shift_right_logical(k * mul0, shift)
                outb[sl] = zv
                zb[sl] = zv
            for i in range(nreal, nvr):
                sl = pl.ds(i * _LANES, _LANES)
                keyb[sl] = neg1
                hb[sl] = dumpv
                valb[sl] = zv
                outb[sl] = zv
                zb[sl] = zv

            for lvl in range(len(_HASH_MULS)):
                last = lvl == len(_HASH_MULS) - 1
                muln = jnp.int32(_HASH_MULS[min(lvl + 1, len(_HASH_MULS) - 1)])

                zchunk = _TSIZE // _NSUB

                def level_body(_muln=muln, _last=last):
                    pltpu.sync_copy(keyb, ktab.at[hb])
                    pltpu.sync_copy(zb.at[pl.ds(0, zchunk)],
                                    vtab.at[pl.ds(s * zchunk, zchunk)])
                    plsc.subcore_barrier()
                    pltpu.sync_copy(ktab.at[hb], gib)

                    def addsrc_body(i, _):
                        for u in range(unroll):
                            sl = pl.ds((i * unroll + u) * _LANES, _LANES)
                            k = keyb[sl]
                            w = (gib[sl] == k) & (k >= 0)
                            addb[sl] = jnp.where(w, valb[sl], zv)
                        return 0

                    lax.fori_loop(0, nvr // unroll, addsrc_body, 0)
                    pltpu.sync_copy(addb, vtab.at[hb], add=True)
                    plsc.subcore_barrier()
                    pltpu.sync_copy(vtab.at[hb], gvb)

                    one16 = jnp.full((_LANES,), 1, jnp.int32)
                    zero16 = jnp.zeros((_LANES,), jnp.int32)

                    def retire_body(i, cnt):
                        for u in range(unroll):
                            sl = pl.ds((i * unroll + u) * _LANES, _LANES)
                            k = keyb[sl]
                            w = (gib[sl] == k) & (k >= 0)
                            outb[sl] = jnp.where(w, gvb[sl], outb[sl])
                            k2 = jnp.where(w, neg1, k)
                            keyb[sl] = k2
                            if not _last:
                                hh = lax.shift_right_logical(k2 * _muln,
                                                             shift)
                                hb[sl] = jnp.where(k2 >= 0, hh, dumpv)
                                cnt = cnt + jnp.where(k2 >= 0, one16, zero16)
                        return cnt

                    cnt = lax.fori_loop(0, nvr // unroll, retire_body, zero16)
                    if not _last:
                        cntv[pl.ds(0, _LANES)] = cnt
                    plsc.subcore_barrier()

                if lvl == 0:
                    level_body()
                else:
                    pltpu.sync_copy(cntv, cntsh.at[pl.ds(s * _LANES, _LANES)])
                    plsc.subcore_barrier()
                    pltpu.sync_copy(cntsh, cntall)
                    acc = jnp.zeros((_LANES,), jnp.int32)
                    for i in range(_NSUB):
                        acc = acc + cntall[pl.ds(i * _LANES, _LANES)]
                    totv[pl.ds(0, _LANES)] = acc
                    av = totv[pl.ds(0, _LANES)]
                    t = av[0]
                    for i in range(1, _LANES):
                        t = t + av[i]
                    pl.when(t > 0)(level_body)

            pltpu.sync_copy(outb.at[pl.ds(0, et)], o_hbm.at[pl.ds(base, et)])

    return kern(col, row, values)


def kernel(embed, edge_index, W1, b1, W2, b2):
    n, d = embed.shape
    e = edge_index.shape[1]
    h = W1.shape[1]
    w1a = W1[:d]
    w1b = W1[d:]
    b1row = b1.reshape(1, h)
    w2row = W2.reshape(1, h)
    b2v = b2.reshape(1, 1)
    col = edge_index[0]
    row = edge_index[1]
    p, q = _mlp_head(embed, w1a, w1b, b1row)
    g = _gather_sum(p, q, col, row, e)
    values = _edge_score(g, w2row, b2v, e)
    return _dedup_sum(col, row, values, e, n)
